# baseline (speedup 1.0000x reference)
import functools

import jax
import jax.numpy as jnp
from jax import lax
from jax.experimental import pallas as pl
from jax.experimental.pallas import tpu as pltpu
from jax.experimental.pallas import tpu_sc as plsc

D_MODEL = 64
NUM_WORKERS = 32
CHUNK = 1024
SUB = 128
N_SUB = CHUNK // SUB


def _make_sc_gather(B):
    b_per_w = B // NUM_WORKERS
    n_chunks = b_per_w // CHUNK
    mesh = plsc.VectorSubcoreMesh(core_axis_name="c", subcore_axis_name="s")

    @functools.partial(
        pl.kernel,
        mesh=mesh,
        out_type=jax.ShapeDtypeStruct((B, D_MODEL), jnp.float32),
        compiler_params=pltpu.CompilerParams(use_tc_tiling_on_sc=False),
        scratch_types=[
            pltpu.VMEM((N_SUB, SUB), jnp.int32),
            pltpu.VMEM((CHUNK, D_MODEL), jnp.float32),
            pltpu.SemaphoreType.DMA,
        ],
    )
    def sc_gather(idx_hbm, table_hbm, out_hbm, idx_v, rows_v, sem):
        wid = lax.axis_index("s") * 2 + lax.axis_index("c")
        base = wid * b_per_w

        def body(i, _):
            off = base + i * CHUNK
            pltpu.sync_copy(
                idx_hbm.at[pl.ds(pl.multiple_of(off // SUB, N_SUB), N_SUB)],
                idx_v,
            )
            copies = []
            for j in range(N_SUB):
                copies.append(
                    pltpu.async_copy(
                        table_hbm.at[idx_v.at[j]],
                        rows_v.at[pl.ds(j * SUB, SUB)],
                        sem,
                    )
                )
            for c in copies:
                c.wait()
            pltpu.sync_copy(
                rows_v, out_hbm.at[pl.ds(pl.multiple_of(off, CHUNK), CHUNK)]
            )
            return 0

        lax.fori_loop(0, n_chunks, body, 0)

    return sc_gather


def kernel(x, table):
    B = x.shape[0] * x.shape[1]
    idx2d = x.reshape(B // SUB, SUB).astype(jnp.int32)
    out = _make_sc_gather(B)(idx2d, table)
    return out.reshape(x.shape[0], x.shape[1], D_MODEL)


# speedup vs baseline: 1.0057x; 1.0057x over previous
import functools

import jax
import jax.numpy as jnp
from jax import lax
from jax.experimental import pallas as pl
from jax.experimental.pallas import tpu as pltpu
from jax.experimental.pallas import tpu_sc as plsc

D_MODEL = 64
NUM_WORKERS = 32
CHUNK = 512
SUB = 128
N_SUB = CHUNK // SUB


def _make_sc_gather(B):
    b_per_w = B // NUM_WORKERS
    n_chunks = b_per_w // CHUNK
    assert n_chunks % 2 == 0 and n_chunks >= 4
    mesh = plsc.VectorSubcoreMesh(core_axis_name="c", subcore_axis_name="s")

    @functools.partial(
        pl.kernel,
        mesh=mesh,
        out_type=jax.ShapeDtypeStruct((B, D_MODEL), jnp.float32),
        compiler_params=pltpu.CompilerParams(use_tc_tiling_on_sc=False),
        scratch_types=[
            pltpu.VMEM((2, N_SUB, SUB), jnp.int32),
            pltpu.VMEM((2, CHUNK, D_MODEL), jnp.float32),
            pltpu.SemaphoreType.DMA,
            pltpu.SemaphoreType.DMA,
            pltpu.SemaphoreType.DMA,
            pltpu.SemaphoreType.DMA,
        ],
    )
    def sc_gather(idx_hbm, table_hbm, out_hbm, idx_v, rows_v,
                  gsem0, gsem1, wsem0, wsem1):
        gsems = (gsem0, gsem1)
        wsems = (wsem0, wsem1)
        wid = lax.axis_index("s") * 2 + lax.axis_index("c")
        base = wid * b_per_w

        def load_and_fire(i, b):
            row = (base + i * CHUNK) // SUB
            pltpu.sync_copy(idx_hbm.at[pl.ds(row, N_SUB)], idx_v.at[b])
            for j in range(N_SUB):
                pltpu.async_copy(
                    table_hbm.at[idx_v.at[b, j]],
                    rows_v.at[b, pl.ds(j * SUB, SUB)],
                    gsems[b],
                )

        def wait_gathers(b):
            for j in range(N_SUB):
                pltpu.make_async_copy(
                    table_hbm.at[idx_v.at[b, j]],
                    rows_v.at[b, pl.ds(j * SUB, SUB)],
                    gsems[b],
                ).wait()

        def fire_write(i, b):
            pltpu.async_copy(
                rows_v.at[b],
                out_hbm.at[pl.ds(base + i * CHUNK, CHUNK)],
                wsems[b],
            )

        def wait_write(b):
            pltpu.make_async_copy(
                rows_v.at[b],
                out_hbm.at[pl.ds(base, CHUNK)],
                wsems[b],
            ).wait()


        load_and_fire(0, 0)
        load_and_fire(1, 1)
        wait_gathers(0)
        fire_write(0, 0)

        def outer(o, _):
            for b, di in ((1, 1), (0, 2)):
                i = 2 * o + di
                nb = 1 - b
                wait_write(nb)
                load_and_fire(i + 1, nb)
                wait_gathers(b)
                fire_write(i, b)
            return 0

        lax.fori_loop(0, (n_chunks - 2) // 2, outer, 0)

        wait_gathers(1)
        fire_write(n_chunks - 1, 1)
        wait_write(0)
        wait_write(1)

    return sc_gather


def kernel(x, table):
    B = x.shape[0] * x.shape[1]
    idx2d = x.reshape(B // SUB, SUB).astype(jnp.int32)
    out = _make_sc_gather(B)(idx2d, table)
    return out.reshape(x.shape[0], x.shape[1], D_MODEL)


# speedup vs baseline: 1.0302x; 1.0244x over previous
import functools

import jax
import jax.numpy as jnp
from jax import lax
from jax.experimental import pallas as pl
from jax.experimental.pallas import tpu as pltpu
from jax.experimental.pallas import tpu_sc as plsc

D_MODEL = 64
NUM_WORKERS = 32
CHUNK = 512
SUB = 128
N_SUB = CHUNK // SUB


def _make_sc_gather(B):
    b_per_w = B // NUM_WORKERS
    n_chunks = b_per_w // CHUNK
    assert n_chunks % 2 == 0 and n_chunks >= 4
    mesh = plsc.VectorSubcoreMesh(core_axis_name="c", subcore_axis_name="s")

    @functools.partial(
        pl.kernel,
        mesh=mesh,
        out_type=jax.ShapeDtypeStruct((B, D_MODEL), jnp.float32),
        compiler_params=pltpu.CompilerParams(use_tc_tiling_on_sc=False),
        scratch_types=[
            pltpu.VMEM((2, N_SUB, SUB), jnp.int32),
            pltpu.VMEM((2, CHUNK, D_MODEL), jnp.float32),
            pltpu.SemaphoreType.DMA,
            pltpu.SemaphoreType.DMA,
            pltpu.SemaphoreType.DMA,
            pltpu.SemaphoreType.DMA,
        ],
    )
    def sc_gather(idx_hbm, table_hbm, out_hbm, idx_v, rows_v,
                  gsem0, gsem1, wsem0, wsem1):
        gsems = (gsem0, gsem1)
        wsems = (wsem0, wsem1)
        wid = lax.axis_index("s") * 2 + lax.axis_index("c")
        base = wid * b_per_w

        def load_and_fire(i, b):
            row = (base + i * CHUNK) // SUB
            pltpu.sync_copy(idx_hbm.at[pl.ds(row, N_SUB)], idx_v.at[b])
            for j in range(N_SUB):
                pltpu.async_copy(
                    table_hbm.at[idx_v.at[b, j]],
                    rows_v.at[b, pl.ds(j * SUB, SUB)],
                    gsems[b],
                )

        def wait_gathers(b):
            for j in range(N_SUB):
                pltpu.make_async_copy(
                    table_hbm.at[idx_v.at[b, j]],
                    rows_v.at[b, pl.ds(j * SUB, SUB)],
                    gsems[b],
                ).wait()

        def fire_write(i, b):
            pltpu.async_copy(
                rows_v.at[b],
                out_hbm.at[pl.ds(base + i * CHUNK, CHUNK)],
                wsems[b],
            )

        def wait_write(b):
            pltpu.make_async_copy(
                rows_v.at[b],
                out_hbm.at[pl.ds(base, CHUNK)],
                wsems[b],
            ).wait()


        load_and_fire(0, 0)
        load_and_fire(1, 1)
        wait_gathers(0)
        fire_write(0, 0)

        def outer(o, _):
            for b, di in ((1, 1), (0, 2)):
                i = 2 * o + di
                nb = 1 - b
                wait_write(nb)
                load_and_fire(i + 1, nb)
                wait_gathers(b)
                fire_write(i, b)
            return 0

        lax.fori_loop(0, (n_chunks - 2) // 2, outer, 0)

        wait_gathers(1)
        fire_write(n_chunks - 1, 1)
        wait_write(0)
        wait_write(1)

    return sc_gather


def kernel(x, table):
    NB, S = x.shape
    B = NB * S
    idx2d = x.T.astype(jnp.int32).reshape(B // SUB, SUB)
    out = _make_sc_gather(B)(idx2d, table)
    return out.reshape(S, NB, D_MODEL).transpose(1, 0, 2)


# speedup vs baseline: 1.0340x; 1.0037x over previous
import functools

import jax
import jax.numpy as jnp
from jax import lax
from jax.experimental import pallas as pl
from jax.experimental.pallas import tpu as pltpu
from jax.experimental.pallas import tpu_sc as plsc

D_MODEL = 64
NUM_WORKERS = 32
CHUNK = 512
SUB = 128
N_SUB = CHUNK // SUB


def _make_sc_gather(B):
    b_per_w = B // NUM_WORKERS
    n_chunks = b_per_w // CHUNK
    assert n_chunks % 2 == 0 and n_chunks >= 4
    mesh = plsc.VectorSubcoreMesh(core_axis_name="c", subcore_axis_name="s")

    @functools.partial(
        pl.kernel,
        mesh=mesh,
        out_type=jax.ShapeDtypeStruct((B, D_MODEL), jnp.float32),
        compiler_params=pltpu.CompilerParams(
            use_tc_tiling_on_sc=False, skip_device_barrier=True
        ),
        scratch_types=[
            pltpu.VMEM((2, N_SUB, SUB), jnp.int32),
            pltpu.VMEM((2, CHUNK, D_MODEL), jnp.float32),
            pltpu.SemaphoreType.DMA,
            pltpu.SemaphoreType.DMA,
            pltpu.SemaphoreType.DMA,
            pltpu.SemaphoreType.DMA,
        ],
    )
    def sc_gather(idx_hbm, table_hbm, out_hbm, idx_v, rows_v,
                  gsem0, gsem1, wsem0, wsem1):
        gsems = (gsem0, gsem1)
        wsems = (wsem0, wsem1)
        wid = lax.axis_index("s") * 2 + lax.axis_index("c")
        base = wid * b_per_w

        def load_and_fire(i, b):
            row = (base + i * CHUNK) // SUB
            pltpu.sync_copy(idx_hbm.at[pl.ds(row, N_SUB)], idx_v.at[b])
            for j in range(N_SUB):
                pltpu.async_copy(
                    table_hbm.at[idx_v.at[b, j]],
                    rows_v.at[b, pl.ds(j * SUB, SUB)],
                    gsems[b],
                )

        def wait_gathers(b):
            for j in range(N_SUB):
                pltpu.make_async_copy(
                    table_hbm.at[idx_v.at[b, j]],
                    rows_v.at[b, pl.ds(j * SUB, SUB)],
                    gsems[b],
                ).wait()

        def fire_write(i, b):
            pltpu.async_copy(
                rows_v.at[b],
                out_hbm.at[pl.ds(base + i * CHUNK, CHUNK)],
                wsems[b],
            )

        def wait_write(b):
            pltpu.make_async_copy(
                rows_v.at[b],
                out_hbm.at[pl.ds(base, CHUNK)],
                wsems[b],
            ).wait()


        load_and_fire(0, 0)
        load_and_fire(1, 1)
        wait_gathers(0)
        fire_write(0, 0)

        def outer(o, _):
            for b, di in ((1, 1), (0, 2)):
                i = 2 * o + di
                nb = 1 - b
                wait_write(nb)
                load_and_fire(i + 1, nb)
                wait_gathers(b)
                fire_write(i, b)
            return 0

        lax.fori_loop(0, (n_chunks - 2) // 2, outer, 0)

        wait_gathers(1)
        fire_write(n_chunks - 1, 1)
        wait_write(0)
        wait_write(1)

    return sc_gather


def kernel(x, table):
    NB, S = x.shape
    B = NB * S
    idx2d = x.T.astype(jnp.int32).reshape(B // SUB, SUB)
    out = _make_sc_gather(B)(idx2d, table)
    return out.reshape(S, NB, D_MODEL).transpose(1, 0, 2)
